# baseline (speedup 1.0000x reference)
import math

import jax
import jax.numpy as jnp
from jax.experimental import pallas as pl
from jax.experimental.pallas import tpu as pltpu

_B, _T, _D = 256, 27, 1024
_N = _B * _T
_BM = 864
_BN = 256
_GM = _N // _BM
_GN = _N // _BN
_BB = 32
_BP = _BB * _T
_LAM = math.log2(math.e)


def _prep_kernel(tgt_ref, q_ref, nv_ref):
    t = tgt_ref[...]
    col = jax.lax.broadcasted_iota(jnp.int32, (_B, _T), 1)
    isz = jnp.logical_and(t == 0, col >= 1)
    posv = jnp.where(isz, col, _T)
    fz = jnp.min(posv, axis=1, keepdims=True)
    fz = jnp.where(fz >= _T, 1, fz)
    valid = col < fz
    q_ref[...] = jnp.where(valid, t, 0).astype(jnp.float32)
    nv_ref[...] = jnp.reshape(jnp.sum(valid.astype(jnp.float32)), (1, 1))


def _norm_kernel(f_ref, fl_ref):
    x = f_ref[...].reshape(_BP, _D)
    nrm = jnp.sqrt(jnp.sum(x * x, axis=1, keepdims=True))
    fl_ref[...] = (x * (1.0 / jnp.maximum(nrm, 1e-12))).astype(jnp.bfloat16)


def _main_kernel(fl_ref, fr_ref, qr_ref, qc_ref, out_ref, spos_ref, sneg_ref):
    j = pl.program_id(1)

    @pl.when(j == 0)
    def _():
        spos_ref[...] = jnp.zeros_like(spos_ref)
        sneg_ref[...] = jnp.zeros_like(sneg_ref)

    fr = fr_ref[...] * jnp.bfloat16(_LAM)
    h = jax.lax.dot_general(fl_ref[...], fr,
                            (((1,), (1,)), ((), ())),
                            preferred_element_type=jnp.float32)
    eh = jnp.exp2(h)
    ehinv = 1.0 / eh
    qr = qr_ref[...]
    qc = qc_ref[...]
    eq = qr == qc
    cv = jnp.where(qc != 0.0, 1.0, 0.0)
    pw = jnp.where(eq, ehinv, 0.0)
    nw = jnp.where(eq, 0.0, eh) * cv
    spos_ref[...] += pw[:, :128] + pw[:, 128:]
    sneg_ref[...] += nw[:, :128] + nw[:, 128:]

    @pl.when(j == _GN - 1)
    def _():
        sp = jnp.sum(spos_ref[...], axis=1, keepdims=True)
        sn = jnp.sum(sneg_ref[...], axis=1, keepdims=True)
        z = jnp.log(jnp.maximum(sp, 1e-30) * jnp.maximum(sn, 1e-30))
        rl = jax.nn.softplus(z)
        mrow = qr_ref[:, 0:1] != 0.0
        total = jnp.sum(jnp.where(mrow, rl, 0.0))
        out_ref[...] = jnp.broadcast_to(jnp.reshape(total, (1, 1, 1)), (1, 1, 128))


def _final_kernel(parts_ref, nv_ref, out_ref):
    out_ref[...] = jnp.reshape(jnp.sum(parts_ref[:, :, 0:1]) / nv_ref[0, 0], (1, 1))


@jax.jit
def kernel(input_f, target):
    q_bt, nv = pl.pallas_call(
        _prep_kernel,
        out_shape=(jax.ShapeDtypeStruct((_B, _T), jnp.float32),
                   jax.ShapeDtypeStruct((1, 1), jnp.float32)),
    )(target)

    qv = q_bt.reshape(_N)
    qrow = jnp.broadcast_to(qv[:, None], (_N, _BN))
    qcol = qv.reshape(1, _N)

    fl = pl.pallas_call(
        _norm_kernel,
        grid=(_B // _BB,),
        in_specs=[pl.BlockSpec((_BB, _T, _D), lambda i: (i, 0, 0))],
        out_specs=pl.BlockSpec((_BP, _D), lambda i: (i, 0)),
        out_shape=jax.ShapeDtypeStruct((_N, _D), jnp.bfloat16),
        compiler_params=pltpu.CompilerParams(
            dimension_semantics=("parallel",)),
    )(input_f)

    parts = pl.pallas_call(
        _main_kernel,
        grid=(_GM, _GN),
        in_specs=[
            pl.BlockSpec((_BM, _D), lambda i, j: (i, 0)),
            pl.BlockSpec((_BN, _D), lambda i, j: (j, 0)),
            pl.BlockSpec((_BM, _BN), lambda i, j: (i, 0)),
            pl.BlockSpec((1, _BN), lambda i, j: (0, j)),
        ],
        out_specs=pl.BlockSpec((1, 1, 128), lambda i, j: (i, 0, 0)),
        out_shape=jax.ShapeDtypeStruct((_GM, 1, 128), jnp.float32),
        scratch_shapes=[pltpu.VMEM((_BM, 128), jnp.float32),
                        pltpu.VMEM((_BM, 128), jnp.float32)],
        compiler_params=pltpu.CompilerParams(
            dimension_semantics=("parallel", "arbitrary"),
            vmem_limit_bytes=64 * 1024 * 1024),
    )(fl, fl, qrow, qcol)

    out = pl.pallas_call(
        _final_kernel,
        out_shape=jax.ShapeDtypeStruct((1, 1), jnp.float32),
    )(parts, nv)
    return out[0, 0]


# speedup vs baseline: 1.2032x; 1.2032x over previous
import math

import jax
import jax.numpy as jnp
from jax.experimental import pallas as pl
from jax.experimental.pallas import tpu as pltpu

_B, _T, _D = 256, 27, 1024
_N = _B * _T
_BM = 1728
_BN = 256
_GM = _N // _BM
_GN = _N // _BN
_BB = 32
_BP = _BB * _T
_LAM = math.log2(math.e)


def _prep_kernel(tgt_ref, q_ref, nv_ref):
    t = tgt_ref[...]
    col = jax.lax.broadcasted_iota(jnp.int32, (_B, _T), 1)
    isz = jnp.logical_and(t == 0, col >= 1)
    posv = jnp.where(isz, col, _T)
    fz = jnp.min(posv, axis=1, keepdims=True)
    fz = jnp.where(fz >= _T, 1, fz)
    valid = col < fz
    q_ref[...] = jnp.where(valid, t, 0).astype(jnp.float32)
    nv_ref[...] = jnp.reshape(jnp.sum(valid.astype(jnp.float32)), (1, 1))


def _norm_kernel(f_ref, fl_ref):
    x = f_ref[...].reshape(_BP, _D)
    nrm = jnp.sqrt(jnp.sum(x * x, axis=1, keepdims=True))
    fl_ref[...] = (x * (1.0 / jnp.maximum(nrm, 1e-12))).astype(jnp.bfloat16)


def _main_kernel(fl_ref, fr_ref, qr_ref, qc_ref, out_ref, spos_ref, sneg_ref):
    j = pl.program_id(1)

    @pl.when(j == 0)
    def _():
        spos_ref[...] = jnp.zeros_like(spos_ref)
        sneg_ref[...] = jnp.zeros_like(sneg_ref)

    fr = fr_ref[...] * jnp.bfloat16(_LAM)
    h = jax.lax.dot_general(fl_ref[...], fr,
                            (((1,), (1,)), ((), ())),
                            preferred_element_type=jnp.float32)
    eh = jnp.exp2(h)
    ehinv = 1.0 / eh
    qr = qr_ref[...]
    qc = qc_ref[...]
    eq = qr == qc
    cv = jnp.where(qc != 0.0, 1.0, 0.0)
    pw = jnp.where(eq, ehinv, 0.0)
    nw = jnp.where(eq, 0.0, eh) * cv
    spos_ref[...] += pw[:, :128] + pw[:, 128:]
    sneg_ref[...] += nw[:, :128] + nw[:, 128:]

    @pl.when(j == _GN - 1)
    def _():
        sp = jnp.sum(spos_ref[...], axis=1, keepdims=True)
        sn = jnp.sum(sneg_ref[...], axis=1, keepdims=True)
        z = jnp.log(jnp.maximum(sp, 1e-30) * jnp.maximum(sn, 1e-30))
        rl = jax.nn.softplus(z)
        mrow = qr_ref[:, 0:1] != 0.0
        total = jnp.sum(jnp.where(mrow, rl, 0.0))
        out_ref[...] = jnp.broadcast_to(jnp.reshape(total, (1, 1, 1)), (1, 1, 128))


def _final_kernel(parts_ref, nv_ref, out_ref):
    out_ref[...] = jnp.reshape(jnp.sum(parts_ref[:, :, 0:1]) / nv_ref[0, 0], (1, 1))


@jax.jit
def kernel(input_f, target):
    q_bt, nv = pl.pallas_call(
        _prep_kernel,
        out_shape=(jax.ShapeDtypeStruct((_B, _T), jnp.float32),
                   jax.ShapeDtypeStruct((1, 1), jnp.float32)),
    )(target)

    qv = q_bt.reshape(_N)
    qrow = jnp.broadcast_to(qv[:, None], (_N, _BN))
    qcol = qv.reshape(1, _N)

    fl = pl.pallas_call(
        _norm_kernel,
        grid=(_B // _BB,),
        in_specs=[pl.BlockSpec((_BB, _T, _D), lambda i: (i, 0, 0))],
        out_specs=pl.BlockSpec((_BP, _D), lambda i: (i, 0)),
        out_shape=jax.ShapeDtypeStruct((_N, _D), jnp.bfloat16),
        compiler_params=pltpu.CompilerParams(
            dimension_semantics=("parallel",)),
    )(input_f)

    parts = pl.pallas_call(
        _main_kernel,
        grid=(_GM, _GN),
        in_specs=[
            pl.BlockSpec((_BM, _D), lambda i, j: (i, 0)),
            pl.BlockSpec((_BN, _D), lambda i, j: (j, 0)),
            pl.BlockSpec((_BM, _BN), lambda i, j: (i, 0)),
            pl.BlockSpec((1, _BN), lambda i, j: (0, j)),
        ],
        out_specs=pl.BlockSpec((1, 1, 128), lambda i, j: (i, 0, 0)),
        out_shape=jax.ShapeDtypeStruct((_GM, 1, 128), jnp.float32),
        scratch_shapes=[pltpu.VMEM((_BM, 128), jnp.float32),
                        pltpu.VMEM((_BM, 128), jnp.float32)],
        compiler_params=pltpu.CompilerParams(
            dimension_semantics=("parallel", "arbitrary"),
            vmem_limit_bytes=64 * 1024 * 1024),
    )(fl, fl, qrow, qcol)

    out = pl.pallas_call(
        _final_kernel,
        out_shape=jax.ShapeDtypeStruct((1, 1), jnp.float32),
    )(parts, nv)
    return out[0, 0]


# speedup vs baseline: 1.2931x; 1.0747x over previous
import math

import jax
import jax.numpy as jnp
from jax.experimental import pallas as pl
from jax.experimental.pallas import tpu as pltpu

_B, _T, _D = 256, 27, 1024
_N = _B * _T
_BM = 3456
_BN = 256
_GM = _N // _BM
_GN = _N // _BN
_BB = 32
_BP = _BB * _T
_LAM = math.log2(math.e)


def _prep_kernel(tgt_ref, q_ref, nv_ref):
    t = tgt_ref[...]
    col = jax.lax.broadcasted_iota(jnp.int32, (_B, _T), 1)
    isz = jnp.logical_and(t == 0, col >= 1)
    posv = jnp.where(isz, col, _T)
    fz = jnp.min(posv, axis=1, keepdims=True)
    fz = jnp.where(fz >= _T, 1, fz)
    valid = col < fz
    q_ref[...] = jnp.where(valid, t, 0).astype(jnp.float32)
    nv_ref[...] = jnp.reshape(jnp.sum(valid.astype(jnp.float32)), (1, 1))


def _norm_kernel(f_ref, fl_ref):
    x = f_ref[...].reshape(_BP, _D)
    nrm = jnp.sqrt(jnp.sum(x * x, axis=1, keepdims=True))
    fl_ref[...] = (x * (1.0 / jnp.maximum(nrm, 1e-12))).astype(jnp.bfloat16)


def _main_kernel(fl_ref, fr_ref, qr_ref, qc_ref, out_ref, spos_ref, sneg_ref):
    j = pl.program_id(1)

    @pl.when(j == 0)
    def _():
        spos_ref[...] = jnp.zeros_like(spos_ref)
        sneg_ref[...] = jnp.zeros_like(sneg_ref)

    fr = fr_ref[...] * jnp.bfloat16(_LAM)
    h = jax.lax.dot_general(fl_ref[...], fr,
                            (((1,), (1,)), ((), ())),
                            preferred_element_type=jnp.float32)
    eh = jnp.exp2(h)
    ehinv = 1.0 / eh
    qr = qr_ref[...]
    qc = qc_ref[...]
    eq = qr == qc
    cv = jnp.where(qc != 0.0, 1.0, 0.0)
    pw = jnp.where(eq, ehinv, 0.0)
    nw = jnp.where(eq, 0.0, eh) * cv
    spos_ref[...] += pw[:, :128] + pw[:, 128:]
    sneg_ref[...] += nw[:, :128] + nw[:, 128:]

    @pl.when(j == _GN - 1)
    def _():
        sp = jnp.sum(spos_ref[...], axis=1, keepdims=True)
        sn = jnp.sum(sneg_ref[...], axis=1, keepdims=True)
        z = jnp.log(jnp.maximum(sp, 1e-30) * jnp.maximum(sn, 1e-30))
        rl = jax.nn.softplus(z)
        mrow = qr_ref[:, 0:1] != 0.0
        total = jnp.sum(jnp.where(mrow, rl, 0.0))
        out_ref[...] = jnp.broadcast_to(jnp.reshape(total, (1, 1, 1)), (1, 1, 128))


def _final_kernel(parts_ref, nv_ref, out_ref):
    out_ref[...] = jnp.reshape(jnp.sum(parts_ref[:, :, 0:1]) / nv_ref[0, 0], (1, 1))


@jax.jit
def kernel(input_f, target):
    q_bt, nv = pl.pallas_call(
        _prep_kernel,
        out_shape=(jax.ShapeDtypeStruct((_B, _T), jnp.float32),
                   jax.ShapeDtypeStruct((1, 1), jnp.float32)),
    )(target)

    qv = q_bt.reshape(_N)
    qrow = jnp.broadcast_to(qv[:, None], (_N, _BN))
    qcol = qv.reshape(1, _N)

    fl = pl.pallas_call(
        _norm_kernel,
        grid=(_B // _BB,),
        in_specs=[pl.BlockSpec((_BB, _T, _D), lambda i: (i, 0, 0))],
        out_specs=pl.BlockSpec((_BP, _D), lambda i: (i, 0)),
        out_shape=jax.ShapeDtypeStruct((_N, _D), jnp.bfloat16),
        compiler_params=pltpu.CompilerParams(
            dimension_semantics=("parallel",)),
    )(input_f)

    parts = pl.pallas_call(
        _main_kernel,
        grid=(_GM, _GN),
        in_specs=[
            pl.BlockSpec((_BM, _D), lambda i, j: (i, 0)),
            pl.BlockSpec((_BN, _D), lambda i, j: (j, 0)),
            pl.BlockSpec((_BM, _BN), lambda i, j: (i, 0)),
            pl.BlockSpec((1, _BN), lambda i, j: (0, j)),
        ],
        out_specs=pl.BlockSpec((1, 1, 128), lambda i, j: (i, 0, 0)),
        out_shape=jax.ShapeDtypeStruct((_GM, 1, 128), jnp.float32),
        scratch_shapes=[pltpu.VMEM((_BM, 128), jnp.float32),
                        pltpu.VMEM((_BM, 128), jnp.float32)],
        compiler_params=pltpu.CompilerParams(
            dimension_semantics=("parallel", "arbitrary"),
            vmem_limit_bytes=64 * 1024 * 1024),
    )(fl, fl, qrow, qcol)

    out = pl.pallas_call(
        _final_kernel,
        out_shape=jax.ShapeDtypeStruct((1, 1), jnp.float32),
    )(parts, nv)
    return out[0, 0]
